# baseline (speedup 1.0000x reference)
import functools

import jax
import jax.numpy as jnp
from jax import lax
from jax.experimental import pallas as pl
from jax.experimental.pallas import tpu as pltpu
from jax.experimental.pallas import tpu_sc as plsc


def _sc_gather(table, idx_flat, n_chunks=4):
    V, D = table.shape
    BF = idx_flat.shape[0]
    info = plsc.get_sparse_core_info()
    NC, NS = info.num_cores, info.num_subcores
    NW = NC * NS
    assert BF % (8 * NW) == 0
    b_per_w = BF // NW
    assert b_per_w % n_chunks == 0
    ch = b_per_w // n_chunks

    mesh = plsc.VectorSubcoreMesh(core_axis_name="c", subcore_axis_name="s")

    @functools.partial(
        pl.kernel,
        mesh=mesh,
        out_type=jax.ShapeDtypeStruct((BF, D), table.dtype),
        scratch_types=[
            pltpu.VMEM((b_per_w,), jnp.int32),
            pltpu.VMEM((2, ch, D), table.dtype),
            pltpu.SemaphoreType.DMA((2,)),
            pltpu.SemaphoreType.DMA((2,)),
        ],
        compiler_params=pltpu.CompilerParams(use_tc_tiling_on_sc=False),
    )
    def k(table_hbm, idx_hbm, out_hbm, idx_v, rows_v, gsem, ssem):
        wid = lax.axis_index("s") * NC + lax.axis_index("c")
        base = wid * b_per_w
        pltpu.sync_copy(idx_hbm.at[pl.ds(base, b_per_w)], idx_v)

        def start_gather(c):
            return pltpu.async_copy(
                table_hbm.at[idx_v.at[pl.ds(c * ch, ch)]],
                rows_v.at[c % 2],
                gsem.at[c % 2],
            )

        gathers = [start_gather(0)]
        scatters = [None, None]
        for c in range(n_chunks):
            gathers[c].wait()
            scatters[c % 2] = pltpu.async_copy(
                rows_v.at[c % 2],
                out_hbm.at[pl.ds(base + c * ch, ch)],
                ssem.at[c % 2],
            )
            if c + 1 < n_chunks:
                if scatters[(c + 1) % 2] is not None:
                    scatters[(c + 1) % 2].wait()
                gathers.append(start_gather(c + 1))
        for s in scatters:
            if s is not None:
                s.wait()

    return k(table, idx_flat)


def _mlp_body(x_ref, w1_ref, b1_ref, w2_ref, b2_ref, o_ref):
    h = jnp.dot(x_ref[...], w1_ref[...], preferred_element_type=jnp.float32)
    h = jnp.maximum(h + b1_ref[...], 0.0)
    z = jnp.dot(h, w2_ref[...], preferred_element_type=jnp.float32)
    z = z + b2_ref[...]
    z = z - jnp.max(z, axis=-1, keepdims=True)
    e = jnp.exp(z)
    o_ref[...] = e / jnp.sum(e, axis=-1, keepdims=True)


def _mlp(x, W1, b1, W2, b2, block_b=512, interpret=False):
    B, F = x.shape
    _, H = W1.shape
    _, O = W2.shape
    return pl.pallas_call(
        _mlp_body,
        grid=(B // block_b,),
        in_specs=[
            pl.BlockSpec((block_b, F), lambda i: (i, 0)),
            pl.BlockSpec((F, H), lambda i: (0, 0)),
            pl.BlockSpec((1, H), lambda i: (0, 0)),
            pl.BlockSpec((H, O), lambda i: (0, 0)),
            pl.BlockSpec((1, O), lambda i: (0, 0)),
        ],
        out_specs=pl.BlockSpec((block_b, O), lambda i: (i, 0)),
        out_shape=jax.ShapeDtypeStruct((B, O), jnp.float32),
        interpret=interpret,
    )(x, W1, b1.reshape(1, -1), W2, b2.reshape(1, -1))


def kernel(inputs, table, W1, b1, W2, b2):
    B, S = inputs.shape
    V, D = table.shape
    idx = inputs.astype(jnp.int32).reshape(-1)
    xflat = _sc_gather(table.astype(jnp.bfloat16), idx)
    x = xflat.reshape(B, S * D)
    return _mlp(x, W1.astype(jnp.bfloat16), b1, W2, b2)


# speedup vs baseline: 1.1796x; 1.1796x over previous
import functools

import jax
import jax.numpy as jnp
from jax import lax
from jax.experimental import pallas as pl
from jax.experimental.pallas import tpu as pltpu
from jax.experimental.pallas import tpu_sc as plsc


def _sc_gather(table, idx_flat, n_chunks=4):
    V, D = table.shape
    BF = idx_flat.shape[0]
    info = plsc.get_sparse_core_info()
    NC, NS = info.num_cores, info.num_subcores
    NW = NC * NS
    assert BF % (8 * NW) == 0
    b_per_w = BF // NW
    assert b_per_w % n_chunks == 0
    ch = b_per_w // n_chunks

    mesh = plsc.VectorSubcoreMesh(core_axis_name="c", subcore_axis_name="s")

    @functools.partial(
        pl.kernel,
        mesh=mesh,
        out_type=jax.ShapeDtypeStruct((BF, D), table.dtype),
        scratch_types=[
            pltpu.VMEM((b_per_w,), jnp.int32),
            pltpu.VMEM((2, ch, D), table.dtype),
            pltpu.SemaphoreType.DMA((2,)),
            pltpu.SemaphoreType.DMA((2,)),
        ],
        compiler_params=pltpu.CompilerParams(use_tc_tiling_on_sc=False),
    )
    def k(table_hbm, idx_hbm, out_hbm, idx_v, rows_v, gsem, ssem):
        wid = lax.axis_index("s") * NC + lax.axis_index("c")
        base = wid * b_per_w
        pltpu.sync_copy(idx_hbm.at[pl.ds(base, b_per_w)], idx_v)

        def start_gather(c):
            return pltpu.async_copy(
                table_hbm.at[idx_v.at[pl.ds(c * ch, ch)]],
                rows_v.at[c % 2],
                gsem.at[c % 2],
            )

        gathers = [start_gather(0)]
        scatters = [None, None]
        for c in range(n_chunks):
            gathers[c].wait()
            scatters[c % 2] = pltpu.async_copy(
                rows_v.at[c % 2],
                out_hbm.at[pl.ds(base + c * ch, ch)],
                ssem.at[c % 2],
            )
            if c + 1 < n_chunks:
                if scatters[(c + 1) % 2] is not None:
                    scatters[(c + 1) % 2].wait()
                gathers.append(start_gather(c + 1))
        for s in scatters:
            if s is not None:
                s.wait()

    return k(table, idx_flat)


def _mlp_body(x_ref, w1_ref, b1_ref, w2_ref, b2_ref, o_ref):
    h = jnp.dot(x_ref[...], w1_ref[...], preferred_element_type=jnp.float32)
    h = jnp.maximum(h + b1_ref[...], 0.0)
    z = jnp.dot(h, w2_ref[...], preferred_element_type=jnp.float32)
    z = z + b2_ref[...]
    z = z - jnp.max(z, axis=-1, keepdims=True)
    e = jnp.exp(z)
    o_ref[...] = e / jnp.sum(e, axis=-1, keepdims=True)


def _mlp(x, W1, b1, W2, b2, block_b=512, interpret=False):
    B, F = x.shape
    _, H = W1.shape
    _, O = W2.shape
    return pl.pallas_call(
        _mlp_body,
        grid=(B // block_b,),
        in_specs=[
            pl.BlockSpec((block_b, F), lambda i: (i, 0)),
            pl.BlockSpec((F, H), lambda i: (0, 0)),
            pl.BlockSpec((1, H), lambda i: (0, 0)),
            pl.BlockSpec((H, O), lambda i: (0, 0)),
            pl.BlockSpec((1, O), lambda i: (0, 0)),
        ],
        out_specs=pl.BlockSpec((block_b, O), lambda i: (i, 0)),
        out_shape=jax.ShapeDtypeStruct((B, O), jnp.float32),
        interpret=interpret,
    )(x, W1, b1.reshape(1, -1), W2, b2.reshape(1, -1))


def kernel(inputs, table, W1, b1, W2, b2):
    B, S = inputs.shape
    V, D = table.shape
    idx = inputs.astype(jnp.int32).reshape(-1)
    xflat = _sc_gather(table, idx)
    x = xflat.reshape(B, S * D)
    return _mlp(x, W1, b1, W2, b2)


# speedup vs baseline: 1.1883x; 1.0074x over previous
import functools

import jax
import jax.numpy as jnp
from jax import lax
from jax.experimental import pallas as pl
from jax.experimental.pallas import tpu as pltpu
from jax.experimental.pallas import tpu_sc as plsc


def _sc_gather(table, idx_flat, n_chunks=4):
    V, D = table.shape
    BF = idx_flat.shape[0]
    info = plsc.get_sparse_core_info()
    NC, NS = info.num_cores, info.num_subcores
    NW = NC * NS
    assert BF % (8 * NW) == 0
    b_per_w = BF // NW
    assert b_per_w % n_chunks == 0
    ch = b_per_w // n_chunks

    mesh = plsc.VectorSubcoreMesh(core_axis_name="c", subcore_axis_name="s")

    @functools.partial(
        pl.kernel,
        mesh=mesh,
        out_type=jax.ShapeDtypeStruct((BF, D), table.dtype),
        scratch_types=[
            pltpu.VMEM((b_per_w,), jnp.int32),
            pltpu.VMEM((2, ch, D), table.dtype),
            pltpu.SemaphoreType.DMA((2,)),
            pltpu.SemaphoreType.DMA((2,)),
        ],
        compiler_params=pltpu.CompilerParams(use_tc_tiling_on_sc=False),
    )
    def k(table_hbm, idx_hbm, out_hbm, idx_v, rows_v, gsem, ssem):
        wid = lax.axis_index("s") * NC + lax.axis_index("c")
        base = wid * b_per_w
        pltpu.sync_copy(idx_hbm.at[pl.ds(base, b_per_w)], idx_v)

        def start_gather(c):
            return pltpu.async_copy(
                table_hbm.at[idx_v.at[pl.ds(c * ch, ch)]],
                rows_v.at[c % 2],
                gsem.at[c % 2],
            )

        gathers = [start_gather(0)]
        scatters = [None, None]
        for c in range(n_chunks):
            gathers[c].wait()
            scatters[c % 2] = pltpu.async_copy(
                rows_v.at[c % 2],
                out_hbm.at[pl.ds(base + c * ch, ch)],
                ssem.at[c % 2],
            )
            if c + 1 < n_chunks:
                if scatters[(c + 1) % 2] is not None:
                    scatters[(c + 1) % 2].wait()
                gathers.append(start_gather(c + 1))
        for s in scatters:
            if s is not None:
                s.wait()

    return k(table, idx_flat)


def _mlp_body(block_b, x_hbm, w1_ref, b1_ref, w2_ref, b2_ref, o_ref, xb, sems):
    i = pl.program_id(0)
    n = pl.num_programs(0)

    def copy(j, slot):
        return pltpu.make_async_copy(
            x_hbm.at[pl.ds(j * block_b, block_b), :], xb.at[slot], sems.at[slot]
        )

    @pl.when(i == 0)
    def _():
        copy(0, 0).start()

    cur = jax.lax.rem(i, 2)
    nxt = jax.lax.rem(i + 1, 2)

    @pl.when(i + 1 < n)
    def _():
        copy(i + 1, nxt).start()

    copy(i, cur).wait()
    x = xb[cur]
    h = jnp.dot(x, w1_ref[...], preferred_element_type=jnp.float32)
    h = jnp.maximum(h + b1_ref[...], 0.0)
    z = jnp.dot(h, w2_ref[...], preferred_element_type=jnp.float32)
    z = z + b2_ref[...]
    z = z - jnp.max(z, axis=-1, keepdims=True)
    e = jnp.exp(z)
    o_ref[...] = e / jnp.sum(e, axis=-1, keepdims=True)


def _mlp(x, W1, b1, W2, b2, block_b=512, interpret=False):
    B, F = x.shape
    _, H = W1.shape
    _, O = W2.shape
    return pl.pallas_call(
        functools.partial(_mlp_body, block_b),
        grid=(B // block_b,),
        in_specs=[
            pl.BlockSpec(memory_space=pltpu.MemorySpace.HBM),
            pl.BlockSpec((F, H), lambda i: (0, 0)),
            pl.BlockSpec((1, H), lambda i: (0, 0)),
            pl.BlockSpec((H, O), lambda i: (0, 0)),
            pl.BlockSpec((1, O), lambda i: (0, 0)),
        ],
        out_specs=pl.BlockSpec((block_b, O), lambda i: (i, 0)),
        out_shape=jax.ShapeDtypeStruct((B, O), jnp.float32),
        scratch_shapes=[
            pltpu.VMEM((2, block_b, F), x.dtype),
            pltpu.SemaphoreType.DMA((2,)),
        ],
        interpret=interpret,
    )(x, W1, b1.reshape(1, -1), W2, b2.reshape(1, -1))


def kernel(inputs, table, W1, b1, W2, b2):
    B, S = inputs.shape
    V, D = table.shape
    idx = inputs.astype(jnp.int32).reshape(-1)
    xflat = _sc_gather(table, idx)
    x = xflat.reshape(B, S * D)
    return _mlp(x, W1, b1, W2, b2)
